# baseline (speedup 1.0000x reference)
import functools

import jax
import jax.numpy as jnp
from jax import lax
from jax.experimental import pallas as pl
from jax.experimental.pallas import tpu as pltpu
from jax.experimental.pallas import tpu_sc as plsc

B = 16384
F = 32


@functools.lru_cache(maxsize=None)
def _make_sc_gather(nc: int, ns: int, b_per_w: int):
  mesh = plsc.VectorSubcoreMesh(core_axis_name="c", subcore_axis_name="s")

  @functools.partial(
      pl.kernel,
      mesh=mesh,
      out_type=(
          jax.ShapeDtypeStruct((B, F), jnp.float32),
          jax.ShapeDtypeStruct((B, F), jnp.float32),
          jax.ShapeDtypeStruct((B, F), jnp.float32),
      ),
      scratch_types=[
          pltpu.VMEM((b_per_w,), jnp.int32),
          pltpu.VMEM((b_per_w,), jnp.int32),
          pltpu.VMEM((b_per_w, F), jnp.float32),
          pltpu.VMEM((b_per_w, F), jnp.float32),
          pltpu.VMEM((b_per_w, F), jnp.float32),
          pltpu.VMEM((b_per_w, F), jnp.float32),
          pltpu.SemaphoreType.DMA,
      ],
      compiler_params=pltpu.CompilerParams(use_tc_tiling_on_sc=False),
  )
  def sc_gather(uidx_hbm, iidx_hbm, gu_hbm, gi_hbm, mu_hbm, mi_hbm,
                gmf_out, mlpu_out, mlpi_out,
                uidx_v, iidx_v, gu_v, gi_v, mu_v, mi_v, sem):
    wid = lax.axis_index("s") * nc + lax.axis_index("c")
    base = wid * b_per_w
    pltpu.sync_copy(uidx_hbm.at[pl.ds(base, b_per_w)], uidx_v)
    pltpu.sync_copy(iidx_hbm.at[pl.ds(base, b_per_w)], iidx_v)
    c1 = pltpu.async_copy(gu_hbm.at[uidx_v], gu_v, sem)
    c2 = pltpu.async_copy(gi_hbm.at[iidx_v], gi_v, sem)
    c3 = pltpu.async_copy(mu_hbm.at[uidx_v], mu_v, sem)
    c4 = pltpu.async_copy(mi_hbm.at[iidx_v], mi_v, sem)
    c1.wait()
    c2.wait()
    c3.wait()
    c4.wait()

    def row(i, carry):
      for j in range(F // 16):
        s = pl.ds(j * 16, 16)
        gu_v[i, s] = gu_v[i, s] * gi_v[i, s]
      return carry

    lax.fori_loop(0, b_per_w, row, 0)

    pltpu.sync_copy(gu_v, gmf_out.at[pl.ds(base, b_per_w)])
    pltpu.sync_copy(mu_v, mlpu_out.at[pl.ds(base, b_per_w)])
    pltpu.sync_copy(mi_v, mlpi_out.at[pl.ds(base, b_per_w)])

  return sc_gather


def _tc_mlp_body(mu_ref, mi_ref, gmf_ref, w1a_ref, w1b_ref, b1_ref,
                 w2_ref, b2_ref, wog_ref, wom_ref, bo_ref, out_ref):
  h = jnp.dot(mu_ref[...], w1a_ref[...], preferred_element_type=jnp.float32)
  h = h + jnp.dot(mi_ref[...], w1b_ref[...], preferred_element_type=jnp.float32)
  h = jnp.maximum(h + b1_ref[...], 0.0)
  h2 = jnp.dot(h, w2_ref[...], preferred_element_type=jnp.float32)
  h2 = jnp.maximum(h2 + b2_ref[...], 0.0)
  out = jnp.sum(gmf_ref[...] * wog_ref[...], axis=1)
  out = out + jnp.sum(h2 * wom_ref[...], axis=1)
  out_ref[...] = out + bo_ref[0]


def _tc_mlp(mlp_u, mlp_i, gmf, W1a, W1b, b1, W2, b2, wo_g, wo_m, bo):
  blk = 2048
  grid = (B // blk,)
  row_spec = pl.BlockSpec((blk, F), lambda i: (i, 0))
  full = lambda shape: pl.BlockSpec(shape, lambda i: tuple(0 for _ in shape))
  return pl.pallas_call(
      _tc_mlp_body,
      grid=grid,
      in_specs=[
          row_spec, row_spec, row_spec,
          full(W1a.shape), full(W1b.shape), full(b1.shape),
          full(W2.shape), full(b2.shape),
          full(wo_g.shape), full(wo_m.shape), full(bo.shape),
      ],
      out_specs=pl.BlockSpec((blk,), lambda i: (i,)),
      out_shape=jax.ShapeDtypeStruct((B,), jnp.float32),
  )(mlp_u, mlp_i, gmf, W1a, W1b, b1, W2, b2, wo_g, wo_m, bo)


@jax.jit
def _neumf(user_idx, item_idx, gmf_user_emb, gmf_item_emb,
           mlp_user_emb, mlp_item_emb, W1, b1, W2, b2, Wo, bo):
  info = plsc.get_sparse_core_info()
  nw = info.num_cores * info.num_subcores
  sc = _make_sc_gather(info.num_cores, info.num_subcores, B // nw)
  gmf, mlp_u, mlp_i = sc(
      user_idx.astype(jnp.int32), item_idx.astype(jnp.int32),
      gmf_user_emb, gmf_item_emb, mlp_user_emb, mlp_item_emb)
  W1a, W1b = W1[:F], W1[F:]
  wo_g, wo_m = Wo[:F, 0], Wo[F:, 0]
  return _tc_mlp(mlp_u, mlp_i, gmf, W1a, W1b, b1, W2, b2, wo_g, wo_m, bo)


def kernel(user_idx, item_idx, gmf_user_emb, gmf_item_emb,
           mlp_user_emb, mlp_item_emb, W1, b1, W2, b2, Wo, bo):
  return _neumf(user_idx, item_idx, gmf_user_emb, gmf_item_emb,
                mlp_user_emb, mlp_item_emb, W1, b1, W2, b2, Wo, bo)
